# baseline (speedup 1.0000x reference)
import functools

import jax
import jax.numpy as jnp
from jax import lax
from jax.experimental import pallas as pl
from jax.experimental.pallas import tpu as pltpu
from jax.experimental.pallas import tpu_sc as plsc

VOCAB = 100000
EMB = 128
NCLASS = 100
B = 4096
L = 200

NCORES = 2
NSUB = 16
NW = NCORES * NSUB
BPW = B // NW
CH = 16
NCHUNK = BPW // CH
LANES = 16
KREG = EMB // LANES


def _sc_mean(x, emb_table):
    mesh = plsc.VectorSubcoreMesh(
        core_axis_name="c", subcore_axis_name="s",
        num_cores=NCORES, num_subcores=NSUB)

    @functools.partial(
        pl.kernel,
        mesh=mesh,
        out_type=jax.ShapeDtypeStruct((B, EMB), jnp.float32),
        scratch_types=[
            pltpu.VMEM((BPW * L,), jnp.int32),
            pltpu.VMEM((4, L, EMB // 2), jnp.int32),
            pltpu.VMEM((BPW, EMB), jnp.float32),
            pltpu.SemaphoreType.DMA,
            pltpu.SemaphoreType.DMA,
            pltpu.SemaphoreType.DMA,
            pltpu.SemaphoreType.DMA,
        ],
        compiler_params=pltpu.CompilerParams(
            needs_layout_passes=False, use_tc_tiling_on_sc=False),
    )
    def k(x_hbm, tab_hbm, mean_hbm, idx_v, rows_v, mst_v,
          sem0, sem1, sem2, sem3):
        sems = (sem0, sem1, sem2, sem3)
        wid = lax.axis_index("s") * NCORES + lax.axis_index("c")
        base = wid * BPW

        def issue(s, buf, sem):
            pltpu.async_copy(
                tab_hbm.at[idx_v.at[pl.ds(s * L, 128)]],
                rows_v.at[buf, pl.ds(0, 128)], sem)
            pltpu.async_copy(
                tab_hbm.at[idx_v.at[pl.ds(s * L + 128, 72)]],
                rows_v.at[buf, pl.ds(128, 72)], sem)

        def wait(buf, sem):
            pltpu.make_async_copy(
                tab_hbm.at[idx_v.at[pl.ds(0, 128)]],
                rows_v.at[buf, pl.ds(0, 128)], sem).wait()
            pltpu.make_async_copy(
                tab_hbm.at[idx_v.at[pl.ds(128, 72)]],
                rows_v.at[buf, pl.ds(128, 72)], sem).wait()

        def consume(s, buf):
            cnt = jnp.zeros((LANES,), jnp.int32)
            for j in range(12):
                v = idx_v[pl.ds(s * L + j * 16, 16)]
                cnt = cnt + plsc.all_reduce_population_count(v != 0)
            lane = lax.iota(jnp.int32, 16)
            vtail = idx_v[pl.ds(s * L + 184, 16)]
            cnt = cnt + plsc.all_reduce_population_count(
                (vtail != 0) & (lane >= 8))
            inv = 1.0 / jnp.maximum(cnt.astype(jnp.float32), 1.0)

            def acc_body(j, acc):
                j0 = j * 8
                for u in range(8):
                    for c in range(4):
                        v = rows_v[buf, j0 + u, c * 16:(c + 1) * 16]
                        lo = plsc.bitcast(v << 16, jnp.float32)
                        hi = plsc.bitcast(v & jnp.int32(-65536), jnp.float32)
                        acc = (acc[:c]
                               + (acc[c] + lo,)
                               + acc[c + 1:4 + c]
                               + (acc[4 + c] + hi,)
                               + acc[5 + c:])
                return acc

            acc0 = tuple(jnp.zeros((LANES,), jnp.float32)
                         for _ in range(KREG))
            acc = lax.fori_loop(0, L // 8, acc_body, acc0)

            for kk in range(KREG):
                mst_v[s, kk * 16:(kk + 1) * 16] = acc[kk] * inv

        pltpu.sync_copy(x_hbm.at[pl.ds(base * L, BPW * L)], idx_v)
        for p in range(3):
            issue(p, p, sems[p])

        def quad_body(t, carry):
            for u in range(4):
                s = t * 4 + u
                wait(u, sems[u])

                @pl.when(s < BPW - 3)
                def _():
                    issue(s + 3, (u + 3) % 4, sems[(u + 3) % 4])

                consume(s, u)
            return carry

        lax.fori_loop(0, BPW // 4, quad_body, 0)
        pltpu.sync_copy(mst_v, mean_hbm.at[pl.ds(base, BPW)])

    return k(x, emb_table)


def _mlp(mean, W1, b1, W2, b2):
    def body(m_ref, w1_ref, b1_ref, w2_ref, b2_ref, o_ref):
        m = m_ref[...]
        h = lax.dot_general(m, w1_ref[...], (((1,), (1,)), ((), ())),
                            preferred_element_type=jnp.float32)
        h = jnp.maximum(h + b1_ref[...], 0.0)
        o = lax.dot_general(h, w2_ref[...], (((1,), (1,)), ((), ())),
                            preferred_element_type=jnp.float32)
        o_ref[...] = o + b2_ref[...]

    nblk = 4
    return pl.pallas_call(
        body,
        out_shape=jax.ShapeDtypeStruct((B, NCLASS), jnp.float32),
        grid=(nblk,),
        in_specs=[
            pl.BlockSpec((B // nblk, EMB), lambda i: (i, 0)),
            pl.BlockSpec((128, EMB), lambda i: (0, 0)),
            pl.BlockSpec((1, 128), lambda i: (0, 0)),
            pl.BlockSpec((NCLASS, 128), lambda i: (0, 0)),
            pl.BlockSpec((1, NCLASS), lambda i: (0, 0)),
        ],
        out_specs=pl.BlockSpec((B // nblk, NCLASS), lambda i: (i, 0)),
    )(mean, W1, b1.reshape(1, 128), W2, b2.reshape(1, NCLASS))


def _pack(tab):
    nb = 50
    rows = VOCAB // nb

    def body(t_ref, o_hbm, o_v, sem):
        i = pl.program_id(0)
        v = lax.bitcast_convert_type(t_ref[...], jnp.int32)
        r = v + ((v >> 16) & 1) + jnp.int32(0x7FFF)
        lo = r[:, : EMB // 2]
        hi = r[:, EMB // 2:]
        o_v[...] = ((lo >> 16) & jnp.int32(0xFFFF)) | (hi & jnp.int32(-65536))
        dma = pltpu.make_async_copy(
            o_v, o_hbm.at[pl.ds(i * rows, rows), :], sem)
        dma.start()
        dma.wait()

    return pl.pallas_call(
        body,
        out_shape=jax.ShapeDtypeStruct((VOCAB, EMB // 2), jnp.int32),
        grid=(nb,),
        in_specs=[pl.BlockSpec((rows, EMB), lambda i: (i, 0))],
        out_specs=pl.BlockSpec(memory_space=pl.ANY),
        scratch_shapes=[
            pltpu.VMEM((rows, EMB // 2), jnp.int32),
            pltpu.SemaphoreType.DMA,
        ],
    )(tab)


def kernel(x, lengths, emb_table, W1, b1, W2, b2):
    tab_i = _pack(emb_table)
    mean = _sc_mean(x.reshape(-1), tab_i)
    return _mlp(mean, W1, b1, W2, b2)


# speedup vs baseline: 1.0755x; 1.0755x over previous
import functools

import jax
import jax.numpy as jnp
from jax import lax
from jax.experimental import pallas as pl
from jax.experimental.pallas import tpu as pltpu
from jax.experimental.pallas import tpu_sc as plsc

VOCAB = 100000
EMB = 128
NCLASS = 100
B = 4096
L = 200

NCORES = 2
NSUB = 16
NW = NCORES * NSUB
BPW = B // NW
CH = 16
NCHUNK = BPW // CH
LANES = 16
KREG = EMB // LANES


def _sc_mean(x, emb_table):
    mesh = plsc.VectorSubcoreMesh(
        core_axis_name="c", subcore_axis_name="s",
        num_cores=NCORES, num_subcores=NSUB)

    @functools.partial(
        pl.kernel,
        mesh=mesh,
        out_type=jax.ShapeDtypeStruct((B, EMB), jnp.float32),
        scratch_types=[
            pltpu.VMEM((BPW * L,), jnp.int32),
            pltpu.VMEM((4, L, EMB // 2), jnp.int32),
            pltpu.VMEM((BPW, EMB), jnp.float32),
            pltpu.SemaphoreType.DMA,
            pltpu.SemaphoreType.DMA,
            pltpu.SemaphoreType.DMA,
            pltpu.SemaphoreType.DMA,
        ],
        compiler_params=pltpu.CompilerParams(
            needs_layout_passes=False, use_tc_tiling_on_sc=False),
    )
    def k(x_hbm, tab_hbm, mean_hbm, idx_v, rows_v, mst_v,
          sem0, sem1, sem2, sem3):
        sems = (sem0, sem1, sem2, sem3)
        wid = lax.axis_index("s") * NCORES + lax.axis_index("c")
        base = wid * BPW

        def issue(s, buf, sem):
            pltpu.async_copy(
                tab_hbm.at[idx_v.at[pl.ds(s * L, 128)]],
                rows_v.at[buf, pl.ds(0, 128)], sem)
            pltpu.async_copy(
                tab_hbm.at[idx_v.at[pl.ds(s * L + 128, 72)]],
                rows_v.at[buf, pl.ds(128, 72)], sem)

        def wait(buf, sem):
            pltpu.make_async_copy(
                tab_hbm.at[idx_v.at[pl.ds(0, 128)]],
                rows_v.at[buf, pl.ds(0, 128)], sem).wait()
            pltpu.make_async_copy(
                tab_hbm.at[idx_v.at[pl.ds(128, 72)]],
                rows_v.at[buf, pl.ds(128, 72)], sem).wait()

        def consume(s, buf):
            cnt = jnp.zeros((LANES,), jnp.int32)
            for j in range(12):
                v = idx_v[pl.ds(s * L + j * 16, 16)]
                cnt = cnt + plsc.all_reduce_population_count(v != 0)
            lane = lax.iota(jnp.int32, 16)
            vtail = idx_v[pl.ds(s * L + 184, 16)]
            cnt = cnt + plsc.all_reduce_population_count(
                (vtail != 0) & (lane >= 8))
            inv = 1.0 / jnp.maximum(cnt.astype(jnp.float32), 1.0)

            def acc_body(j, acc):
                j0 = j * 8
                for u in range(8):
                    for c in range(4):
                        v = rows_v[buf, j0 + u, c * 16:(c + 1) * 16]
                        lo = plsc.bitcast(v << 16, jnp.float32)
                        hi = plsc.bitcast(v & jnp.int32(-65536), jnp.float32)
                        acc = (acc[:c]
                               + (acc[c] + lo,)
                               + acc[c + 1:4 + c]
                               + (acc[4 + c] + hi,)
                               + acc[5 + c:])
                return acc

            acc0 = tuple(jnp.zeros((LANES,), jnp.float32)
                         for _ in range(KREG))
            acc = lax.fori_loop(0, L // 8, acc_body, acc0)

            for kk in range(KREG):
                mst_v[s, kk * 16:(kk + 1) * 16] = acc[kk] * inv

        pltpu.sync_copy(x_hbm.at[pl.ds(base * L, BPW * L)], idx_v)
        for p in range(3):
            issue(p, p, sems[p])

        def quad_body(t, carry):
            for u in range(4):
                s = t * 4 + u
                wait(u, sems[u])

                @pl.when(s < BPW - 3)
                def _():
                    issue(s + 3, (u + 3) % 4, sems[(u + 3) % 4])

                consume(s, u)
            return carry

        lax.fori_loop(0, BPW // 4, quad_body, 0)
        pltpu.sync_copy(mst_v, mean_hbm.at[pl.ds(base, BPW)])

    return k(x, emb_table)


def _mlp(mean, W1, b1, W2, b2):
    def body(m_ref, w1_ref, b1_ref, w2_ref, b2_ref, o_ref):
        m = m_ref[...]
        h = lax.dot_general(m, w1_ref[...], (((1,), (1,)), ((), ())),
                            preferred_element_type=jnp.float32)
        h = jnp.maximum(h + b1_ref[...], 0.0)
        o = lax.dot_general(h, w2_ref[...], (((1,), (1,)), ((), ())),
                            preferred_element_type=jnp.float32)
        o_ref[...] = o + b2_ref[...]

    nblk = 4
    return pl.pallas_call(
        body,
        out_shape=jax.ShapeDtypeStruct((B, NCLASS), jnp.float32),
        grid=(nblk,),
        in_specs=[
            pl.BlockSpec((B // nblk, EMB), lambda i: (i, 0)),
            pl.BlockSpec((128, EMB), lambda i: (0, 0)),
            pl.BlockSpec((1, 128), lambda i: (0, 0)),
            pl.BlockSpec((NCLASS, 128), lambda i: (0, 0)),
            pl.BlockSpec((1, NCLASS), lambda i: (0, 0)),
        ],
        out_specs=pl.BlockSpec((B // nblk, NCLASS), lambda i: (i, 0)),
    )(mean, W1, b1.reshape(1, 128), W2, b2.reshape(1, NCLASS))


def _pack(tab):
    nb = 50
    rows = VOCAB // nb

    def body(t_ref, o_hbm, o_v, sem):
        i = pl.program_id(0)

        def step(buf):
            @pl.when(i >= 2)
            def _():
                pltpu.make_async_copy(
                    o_v.at[buf],
                    o_hbm.at[pl.ds((i - 2) * rows, rows), :], sem).wait()
            v = lax.bitcast_convert_type(t_ref[...], jnp.int32)
            r = v + ((v >> 16) & 1) + jnp.int32(0x7FFF)
            lo = r[:, : EMB // 2]
            hi = r[:, EMB // 2:]
            o_v[buf] = ((lo >> 16) & jnp.int32(0xFFFF)) | (hi & jnp.int32(-65536))
            pltpu.make_async_copy(
                o_v.at[buf], o_hbm.at[pl.ds(i * rows, rows), :], sem).start()

        @pl.when(i % 2 == 0)
        def _():
            step(0)

        @pl.when(i % 2 == 1)
        def _():
            step(1)

        @pl.when(i == nb - 1)
        def _():
            for buf in range(2):
                pltpu.make_async_copy(
                    o_v.at[buf],
                    o_hbm.at[pl.ds((i - 1 + buf) * rows, rows), :], sem).wait()

    return pl.pallas_call(
        body,
        out_shape=jax.ShapeDtypeStruct((VOCAB, EMB // 2), jnp.int32),
        grid=(nb,),
        in_specs=[pl.BlockSpec((rows, EMB), lambda i: (i, 0))],
        out_specs=pl.BlockSpec(memory_space=pl.ANY),
        scratch_shapes=[
            pltpu.VMEM((2, rows, EMB // 2), jnp.int32),
            pltpu.SemaphoreType.DMA,
        ],
    )(tab)


def kernel(x, lengths, emb_table, W1, b1, W2, b2):
    tab_i = _pack(emb_table)
    mean = _sc_mean(x.reshape(-1), tab_i)
    return _mlp(mean, W1, b1, W2, b2)


# speedup vs baseline: 1.1460x; 1.0656x over previous
import functools

import jax
import jax.numpy as jnp
from jax import lax
from jax.experimental import pallas as pl
from jax.experimental.pallas import tpu as pltpu
from jax.experimental.pallas import tpu_sc as plsc

VOCAB = 100000
EMB = 128
NCLASS = 100
B = 4096
L = 200

NCORES = 2
NSUB = 16
NW = NCORES * NSUB
BPW = B // NW
CH = 16
NCHUNK = BPW // CH
LANES = 16
KREG = EMB // LANES


def _sc_mean(x, emb_table):
    mesh = plsc.VectorSubcoreMesh(
        core_axis_name="c", subcore_axis_name="s",
        num_cores=NCORES, num_subcores=NSUB)

    @functools.partial(
        pl.kernel,
        mesh=mesh,
        out_type=jax.ShapeDtypeStruct((B, EMB), jnp.float32),
        scratch_types=[
            pltpu.VMEM((BPW * L,), jnp.int32),
            pltpu.VMEM((4, L, EMB // 2), jnp.int32),
            pltpu.VMEM((BPW, EMB), jnp.float32),
            pltpu.SemaphoreType.DMA,
            pltpu.SemaphoreType.DMA,
            pltpu.SemaphoreType.DMA,
            pltpu.SemaphoreType.DMA,
        ],
        compiler_params=pltpu.CompilerParams(
            needs_layout_passes=False, use_tc_tiling_on_sc=False),
    )
    def k(x_hbm, tab_hbm, mean_hbm, idx_v, rows_v, mst_v,
          sem0, sem1, sem2, sem3):
        sems = (sem0, sem1, sem2, sem3)
        wid = lax.axis_index("s") * NCORES + lax.axis_index("c")
        base = wid * BPW

        def issue(s, buf, sem):
            pltpu.async_copy(
                tab_hbm.at[idx_v.at[pl.ds(s * L, 128)]],
                rows_v.at[buf, pl.ds(0, 128)], sem)
            pltpu.async_copy(
                tab_hbm.at[idx_v.at[pl.ds(s * L + 128, 72)]],
                rows_v.at[buf, pl.ds(128, 72)], sem)

        def wait(buf, sem):
            pltpu.make_async_copy(
                tab_hbm.at[idx_v.at[pl.ds(0, 128)]],
                rows_v.at[buf, pl.ds(0, 128)], sem).wait()
            pltpu.make_async_copy(
                tab_hbm.at[idx_v.at[pl.ds(128, 72)]],
                rows_v.at[buf, pl.ds(128, 72)], sem).wait()

        def consume(s, buf):
            cnt = jnp.zeros((LANES,), jnp.int32)
            for j in range(12):
                v = idx_v[pl.ds(s * L + j * 16, 16)]
                cnt = cnt + plsc.all_reduce_population_count(v != 0)
            lane = lax.iota(jnp.int32, 16)
            vtail = idx_v[pl.ds(s * L + 184, 16)]
            cnt = cnt + plsc.all_reduce_population_count(
                (vtail != 0) & (lane >= 8))
            inv = 1.0 / jnp.maximum(cnt.astype(jnp.float32), 1.0)

            def acc_body(j, acc):
                j0 = j * 8
                for u in range(8):
                    for c in range(4):
                        v = rows_v[buf, j0 + u, c * 16:(c + 1) * 16]
                        lo = plsc.bitcast(v << 16, jnp.float32)
                        hi = plsc.bitcast(v & jnp.int32(-65536), jnp.float32)
                        acc = (acc[:c]
                               + (acc[c] + lo,)
                               + acc[c + 1:4 + c]
                               + (acc[4 + c] + hi,)
                               + acc[5 + c:])
                return acc

            acc0 = tuple(jnp.zeros((LANES,), jnp.float32)
                         for _ in range(KREG))
            acc = lax.fori_loop(0, L // 8, acc_body, acc0)

            for kk in range(KREG):
                mst_v[s, kk * 16:(kk + 1) * 16] = acc[kk] * inv

        pltpu.sync_copy(x_hbm.at[pl.ds(base * L, BPW * L)], idx_v)
        for p in range(3):
            issue(p, p, sems[p])

        def quad_body(t, carry):
            for u in range(4):
                s = t * 4 + u
                wait(u, sems[u])

                @pl.when(s < BPW - 3)
                def _():
                    issue(s + 3, (u + 3) % 4, sems[(u + 3) % 4])

                consume(s, u)
            return carry

        lax.fori_loop(0, BPW // 4, quad_body, 0)
        pltpu.sync_copy(mst_v, mean_hbm.at[pl.ds(base, BPW)])

    return k(x, emb_table)


def _mlp(mean, W1, b1, W2, b2):
    def body(m_ref, w1_ref, b1_ref, w2_ref, b2_ref, o_ref):
        m = m_ref[...]
        h = lax.dot_general(m, w1_ref[...], (((1,), (1,)), ((), ())),
                            preferred_element_type=jnp.float32)
        h = jnp.maximum(h + b1_ref[...], 0.0)
        o = lax.dot_general(h, w2_ref[...], (((1,), (1,)), ((), ())),
                            preferred_element_type=jnp.float32)
        o_ref[...] = o + b2_ref[...]

    nblk = 4
    return pl.pallas_call(
        body,
        out_shape=jax.ShapeDtypeStruct((B, NCLASS), jnp.float32),
        grid=(nblk,),
        in_specs=[
            pl.BlockSpec((B // nblk, EMB), lambda i: (i, 0)),
            pl.BlockSpec((128, EMB), lambda i: (0, 0)),
            pl.BlockSpec((1, 128), lambda i: (0, 0)),
            pl.BlockSpec((NCLASS, 128), lambda i: (0, 0)),
            pl.BlockSpec((1, NCLASS), lambda i: (0, 0)),
        ],
        out_specs=pl.BlockSpec((B // nblk, NCLASS), lambda i: (i, 0)),
    )(mean, W1, b1.reshape(1, 128), W2, b2.reshape(1, NCLASS))


def _pack(tab):
    nb = 25
    rows = VOCAB // nb

    def body(t_ref, o_hbm, o_v, sem):
        i = pl.program_id(0)

        def step(buf):
            @pl.when(i >= 2)
            def _():
                pltpu.make_async_copy(
                    o_v.at[buf],
                    o_hbm.at[pl.ds((i - 2) * rows, rows), :], sem).wait()
            v = lax.bitcast_convert_type(t_ref[...], jnp.int32)
            r = v + ((v >> 16) & 1) + jnp.int32(0x7FFF)
            lo = r[:, : EMB // 2]
            hi = r[:, EMB // 2:]
            o_v[buf] = ((lo >> 16) & jnp.int32(0xFFFF)) | (hi & jnp.int32(-65536))
            pltpu.make_async_copy(
                o_v.at[buf], o_hbm.at[pl.ds(i * rows, rows), :], sem).start()

        @pl.when(i % 2 == 0)
        def _():
            step(0)

        @pl.when(i % 2 == 1)
        def _():
            step(1)

        @pl.when(i == nb - 1)
        def _():
            for buf in range(2):
                pltpu.make_async_copy(
                    o_v.at[buf],
                    o_hbm.at[pl.ds((i - 1 + buf) * rows, rows), :], sem).wait()

    return pl.pallas_call(
        body,
        out_shape=jax.ShapeDtypeStruct((VOCAB, EMB // 2), jnp.int32),
        grid=(nb,),
        in_specs=[pl.BlockSpec((rows, EMB), lambda i: (i, 0))],
        out_specs=pl.BlockSpec(memory_space=pl.ANY),
        scratch_shapes=[
            pltpu.VMEM((2, rows, EMB // 2), jnp.int32),
            pltpu.SemaphoreType.DMA,
        ],
    )(tab)


def kernel(x, lengths, emb_table, W1, b1, W2, b2):
    tab_i = _pack(emb_table)
    mean = _sc_mean(x.reshape(-1), tab_i)
    return _mlp(mean, W1, b1, W2, b2)


# speedup vs baseline: 1.1792x; 1.0290x over previous
import functools

import jax
import jax.numpy as jnp
from jax import lax
from jax.experimental import pallas as pl
from jax.experimental.pallas import tpu as pltpu
from jax.experimental.pallas import tpu_sc as plsc

VOCAB = 100000
EMB = 128
NCLASS = 100
B = 4096
L = 200

NCORES = 2
NSUB = 16
NW = NCORES * NSUB
BPW = B // NW
CH = 16
NCHUNK = BPW // CH
LANES = 16
KREG = EMB // LANES


def _sc_mean(x, emb_table):
    mesh = plsc.VectorSubcoreMesh(
        core_axis_name="c", subcore_axis_name="s",
        num_cores=NCORES, num_subcores=NSUB)

    @functools.partial(
        pl.kernel,
        mesh=mesh,
        out_type=jax.ShapeDtypeStruct((B, EMB), jnp.float32),
        scratch_types=[
            pltpu.VMEM((BPW * L,), jnp.int32),
            pltpu.VMEM((4, L, EMB // 2), jnp.int32),
            pltpu.VMEM((BPW, EMB), jnp.float32),
            pltpu.SemaphoreType.DMA,
            pltpu.SemaphoreType.DMA,
            pltpu.SemaphoreType.DMA,
            pltpu.SemaphoreType.DMA,
        ],
        compiler_params=pltpu.CompilerParams(
            needs_layout_passes=False, use_tc_tiling_on_sc=False),
    )
    def k(x_hbm, tab_hbm, mean_hbm, idx_v, rows_v, mst_v,
          sem0, sem1, sem2, sem3):
        sems = (sem0, sem1, sem2, sem3)
        wid = lax.axis_index("s") * NCORES + lax.axis_index("c")
        base = wid * BPW

        def issue(s, buf, sem):
            pltpu.async_copy(
                tab_hbm.at[idx_v.at[pl.ds(s * L, 128)]],
                rows_v.at[buf, pl.ds(0, 128)], sem)
            pltpu.async_copy(
                tab_hbm.at[idx_v.at[pl.ds(s * L + 128, 72)]],
                rows_v.at[buf, pl.ds(128, 72)], sem)

        def wait(buf, sem):
            pltpu.make_async_copy(
                tab_hbm.at[idx_v.at[pl.ds(0, 128)]],
                rows_v.at[buf, pl.ds(0, 128)], sem).wait()
            pltpu.make_async_copy(
                tab_hbm.at[idx_v.at[pl.ds(128, 72)]],
                rows_v.at[buf, pl.ds(128, 72)], sem).wait()

        def consume(s, buf):
            cnt = jnp.zeros((LANES,), jnp.int32)
            for j in range(12):
                v = idx_v[pl.ds(s * L + j * 16, 16)]
                cnt = cnt + plsc.all_reduce_population_count(v != 0)
            lane = lax.iota(jnp.int32, 16)
            vtail = idx_v[pl.ds(s * L + 184, 16)]
            cnt = cnt + plsc.all_reduce_population_count(
                (vtail != 0) & (lane >= 8))
            inv = 1.0 / jnp.maximum(cnt.astype(jnp.float32), 1.0)

            def acc_body(j, acc):
                j0 = j * 8
                for u in range(8):
                    for c in range(4):
                        v = rows_v[buf, j0 + u, c * 16:(c + 1) * 16]
                        lo = plsc.bitcast(v << 16, jnp.float32)
                        hi = plsc.bitcast(v & jnp.int32(-65536), jnp.float32)
                        acc = (acc[:c]
                               + (acc[c] + lo,)
                               + acc[c + 1:4 + c]
                               + (acc[4 + c] + hi,)
                               + acc[5 + c:])
                return acc

            acc0 = tuple(jnp.zeros((LANES,), jnp.float32)
                         for _ in range(KREG))
            acc = lax.fori_loop(0, L // 8, acc_body, acc0)

            for kk in range(KREG):
                mst_v[s, kk * 16:(kk + 1) * 16] = acc[kk] * inv

        pltpu.sync_copy(x_hbm.at[pl.ds(base * L, BPW * L)], idx_v)
        for p in range(3):
            issue(p, p, sems[p])

        def quad_body(t, carry):
            for u in range(4):
                s = t * 4 + u
                wait(u, sems[u])

                @pl.when(s < BPW - 3)
                def _():
                    issue(s + 3, (u + 3) % 4, sems[(u + 3) % 4])

                consume(s, u)
            return carry

        lax.fori_loop(0, BPW // 4, quad_body, 0)
        pltpu.sync_copy(mst_v, mean_hbm.at[pl.ds(base, BPW)])

    return k(x, emb_table)


def _mlp(mean, W1, b1, W2, b2):
    def body(m_ref, w1_ref, b1_ref, w2_ref, b2_ref, o_ref):
        m = m_ref[...]
        h = lax.dot_general(m, w1_ref[...], (((1,), (1,)), ((), ())),
                            preferred_element_type=jnp.float32)
        h = jnp.maximum(h + b1_ref[...], 0.0)
        o = lax.dot_general(h, w2_ref[...], (((1,), (1,)), ((), ())),
                            preferred_element_type=jnp.float32)
        o_ref[...] = o + b2_ref[...]

    nblk = 4
    return pl.pallas_call(
        body,
        out_shape=jax.ShapeDtypeStruct((B, NCLASS), jnp.float32),
        grid=(nblk,),
        in_specs=[
            pl.BlockSpec((B // nblk, EMB), lambda i: (i, 0)),
            pl.BlockSpec((128, EMB), lambda i: (0, 0)),
            pl.BlockSpec((1, 128), lambda i: (0, 0)),
            pl.BlockSpec((NCLASS, 128), lambda i: (0, 0)),
            pl.BlockSpec((1, NCLASS), lambda i: (0, 0)),
        ],
        out_specs=pl.BlockSpec((B // nblk, NCLASS), lambda i: (i, 0)),
    )(mean, W1, b1.reshape(1, 128), W2, b2.reshape(1, NCLASS))


def _pack(tab):
    nb = 10
    rows = VOCAB // nb

    def body(t_ref, o_hbm, o_v, sem):
        i = pl.program_id(0)

        def step(buf):
            @pl.when(i >= 2)
            def _():
                pltpu.make_async_copy(
                    o_v.at[buf],
                    o_hbm.at[pl.ds((i - 2) * rows, rows), :], sem).wait()
            v = lax.bitcast_convert_type(t_ref[...], jnp.int32)
            r = v + ((v >> 16) & 1) + jnp.int32(0x7FFF)
            lo = r[:, : EMB // 2]
            hi = r[:, EMB // 2:]
            o_v[buf] = ((lo >> 16) & jnp.int32(0xFFFF)) | (hi & jnp.int32(-65536))
            pltpu.make_async_copy(
                o_v.at[buf], o_hbm.at[pl.ds(i * rows, rows), :], sem).start()

        @pl.when(i % 2 == 0)
        def _():
            step(0)

        @pl.when(i % 2 == 1)
        def _():
            step(1)

        @pl.when(i == nb - 1)
        def _():
            for buf in range(2):
                pltpu.make_async_copy(
                    o_v.at[buf],
                    o_hbm.at[pl.ds((i - 1 + buf) * rows, rows), :], sem).wait()

    return pl.pallas_call(
        body,
        out_shape=jax.ShapeDtypeStruct((VOCAB, EMB // 2), jnp.int32),
        grid=(nb,),
        in_specs=[pl.BlockSpec((rows, EMB), lambda i: (i, 0))],
        out_specs=pl.BlockSpec(memory_space=pl.ANY),
        scratch_shapes=[
            pltpu.VMEM((2, rows, EMB // 2), jnp.int32),
            pltpu.SemaphoreType.DMA,
        ],
    )(tab)


def kernel(x, lengths, emb_table, W1, b1, W2, b2):
    tab_i = _pack(emb_table)
    mean = _sc_mean(x.reshape(-1), tab_i)
    return _mlp(mean, W1, b1, W2, b2)
